# baseline (speedup 1.0000x reference)
import functools

import jax
import jax.numpy as jnp
from jax import lax
from jax.experimental import pallas as pl
from jax.experimental.pallas import tpu as pltpu
from jax.experimental.pallas import tpu_sc as plsc

_CHUNK = 128
_NBUF = 5
_W = 128


@functools.partial(jax.jit, static_argnums=(0,))
def _run(total_rows, ids_flat, table_pad, off16):
    D = _W // 2
    info = plsc.get_sparse_core_info()
    NC, NS, L = info.num_cores, info.num_subcores, info.num_lanes
    NW = NC * NS
    rows_per_w = total_rows // NW
    n_chunks = rows_per_w // _CHUNK
    mesh = plsc.VectorSubcoreMesh(core_axis_name="c", subcore_axis_name="s")

    @functools.partial(
        pl.kernel,
        mesh=mesh,
        out_type=jax.ShapeDtypeStruct((total_rows, _W), jnp.float32),
        compiler_params=pltpu.CompilerParams(
            use_tc_tiling_on_sc=False, skip_device_barrier=True
        ),
        scratch_types=[
            pltpu.VMEM((rows_per_w,), jnp.int32),
            pltpu.VMEM((L,), jnp.int32),
            pltpu.VMEM((_NBUF, _CHUNK, _W // 2), jnp.float32),
            pltpu.SemaphoreType.DMA,
            pltpu.SemaphoreType.DMA,
        ],
    )
    def k(ids_hbm, table_hbm, off_hbm, out_hbm, idx_v, off_v, rows_v, sem_g, sem_s):
        wid = lax.axis_index("s") * NC + lax.axis_index("c")
        base_row = wid * rows_per_w
        pltpu.sync_copy(ids_hbm.at[pl.ds(base_row, rows_per_w)], idx_v)
        pltpu.sync_copy(off_hbm, off_v)
        off = off_v[...]

        def shift_ids(j):
            for kk in range(_CHUNK // L):
                sl = pl.ds(j * _CHUNK + kk * L, L)
                idx_v[sl] = idx_v[sl] + off

        def fire_gather(j, b):
            pltpu.async_copy(
                table_hbm.at[idx_v.at[pl.ds(j * _CHUNK, _CHUNK)]],
                rows_v.at[b], sem_g,
            )

        def wait_gather(b):
            pltpu.make_async_copy(
                table_hbm.at[idx_v.at[pl.ds(0, _CHUNK)]],
                rows_v.at[b], sem_g,
            ).wait()

        def fire_scatter(j, b):
            pltpu.async_copy(
                rows_v.at[b],
                out_hbm.at[pl.ds(base_row + j * _CHUNK, _CHUNK), pl.ds(0, D)],
                sem_s,
            )

        def wait_scatter(b):
            pltpu.make_async_copy(
                out_hbm.at[pl.ds(base_row, _CHUNK), pl.ds(0, D)],
                rows_v.at[b], sem_s,
            ).wait()

        for b in range(_NBUF):
            shift_ids(b)
            fire_gather(b, b)

        def body(j0, carry):
            for b in range(_NBUF):
                j = j0 * _NBUF + b
                wait_gather(b)
                fire_scatter(j, b)

                @pl.when(j + _NBUF < n_chunks)
                def _():
                    shift_ids(j + _NBUF)
                    wait_scatter(b)
                    fire_gather(j + _NBUF, b)

            return carry

        lax.fori_loop(0, n_chunks // _NBUF, body, 0)
        for b in range(_NBUF):
            wait_scatter(b)

    return k(ids_flat, table_pad, off16)


def kernel(input_ids, table, offsets):
    B, T, H = input_ids.shape
    N, D = table.shape
    total = B * T * H
    ids_flat = input_ids.astype(jnp.int32).reshape(total)
    off16 = jnp.tile(offsets.astype(jnp.int32), 16 // H)
    tab128 = lax.optimization_barrier(table.reshape(N // 2, 2 * D))
    tab_lin = tab128.reshape(N, D)
    out_pad = _run(total, ids_flat, tab_lin, off16)
    return lax.slice(out_pad, (0, 0), (total, D)).reshape(B, T, H, D)


# speedup vs baseline: 1.0207x; 1.0207x over previous
import functools

import jax
import jax.numpy as jnp
from jax import lax
from jax.experimental import pallas as pl
from jax.experimental.pallas import tpu as pltpu
from jax.experimental.pallas import tpu_sc as plsc

_CHUNK = 128
_NBUF = 5
_W = 128


@functools.partial(jax.jit, static_argnums=(0,))
def _run(total_rows, ids_flat, table_pad, off16):
    D = _W // 2
    info = plsc.get_sparse_core_info()
    NC, NS, L = info.num_cores, info.num_subcores, info.num_lanes
    NW = NC * NS
    rows_per_w = total_rows // NW
    n_chunks = rows_per_w // _CHUNK
    mesh = plsc.VectorSubcoreMesh(core_axis_name="c", subcore_axis_name="s")

    @functools.partial(
        pl.kernel,
        mesh=mesh,
        out_type=jax.ShapeDtypeStruct((total_rows, _W), jnp.float32),
        compiler_params=pltpu.CompilerParams(
            use_tc_tiling_on_sc=False, skip_device_barrier=True
        ),
        scratch_types=[
            pltpu.VMEM((rows_per_w,), jnp.int32),
            pltpu.VMEM((L,), jnp.int32),
            pltpu.VMEM((_NBUF, _CHUNK, _W), jnp.float32),
            pltpu.SemaphoreType.DMA,
            pltpu.SemaphoreType.DMA,
        ],
    )
    def k(ids_hbm, table_hbm, off_hbm, out_hbm, idx_v, off_v, rows_v, sem_g, sem_s):
        wid = lax.axis_index("s") * NC + lax.axis_index("c")
        base_row = wid * rows_per_w
        pltpu.sync_copy(ids_hbm.at[pl.ds(base_row, rows_per_w)], idx_v)
        pltpu.sync_copy(off_hbm, off_v)
        off = off_v[...]

        def shift_ids(j):
            for kk in range(_CHUNK // L):
                sl = pl.ds(j * _CHUNK + kk * L, L)
                idx_v[sl] = idx_v[sl] + off

        def fire_gather(j, b):
            pltpu.async_copy(
                table_hbm.at[idx_v.at[pl.ds(j * _CHUNK, _CHUNK)]],
                rows_v.at[b], sem_g,
            )

        def wait_gather(b):
            pltpu.make_async_copy(
                table_hbm.at[idx_v.at[pl.ds(0, _CHUNK)]],
                rows_v.at[b], sem_g,
            ).wait()

        def fire_scatter(j, b):
            pltpu.async_copy(
                rows_v.at[b, slice(None), pl.ds(0, D)],
                out_hbm.at[pl.ds(base_row + j * _CHUNK, _CHUNK), pl.ds(0, D)],
                sem_s,
            )

        def wait_scatter(b):
            pltpu.make_async_copy(
                out_hbm.at[pl.ds(base_row, _CHUNK), pl.ds(0, D)],
                rows_v.at[b, slice(None), pl.ds(0, D)], sem_s,
            ).wait()

        for b in range(_NBUF):
            shift_ids(b)
            fire_gather(b, b)

        def body(j0, carry):
            for b in range(_NBUF):
                j = j0 * _NBUF + b
                wait_gather(b)
                fire_scatter(j, b)

                @pl.when(j + _NBUF < n_chunks)
                def _():
                    shift_ids(j + _NBUF)
                    wait_scatter(b)
                    fire_gather(j + _NBUF, b)

            return carry

        lax.fori_loop(0, n_chunks // _NBUF, body, 0)
        for b in range(_NBUF):
            wait_scatter(b)

    return k(ids_flat, table_pad, off16)


def kernel(input_ids, table, offsets):
    B, T, H = input_ids.shape
    N, D = table.shape
    total = B * T * H
    ids_flat = input_ids.astype(jnp.int32).reshape(total)
    off16 = jnp.tile(offsets.astype(jnp.int32), 16 // H)
    table_pad = jnp.pad(table, ((0, 0), (0, _W - D)))
    out_pad = _run(total, ids_flat, table_pad, off16)
    return lax.slice(out_pad, (0, 0), (total, D)).reshape(B, T, H, D)
